# baseline (speedup 1.0000x reference)
import functools

import jax
import jax.numpy as jnp
from jax import lax
from jax.experimental import pallas as pl
from jax.experimental.pallas import tpu as pltpu
from jax.experimental.pallas import tpu_sc as plsc


def _sc_gather(xs, weight):
    B = xs.shape[0]
    V, D = weight.shape
    info = plsc.get_sparse_core_info()
    nc, ns = info.num_cores, info.num_subcores
    nw = nc * ns
    b_per_w = B // nw

    mesh = plsc.VectorSubcoreMesh(core_axis_name="c", subcore_axis_name="s")

    @functools.partial(
        pl.kernel,
        mesh=mesh,
        out_type=jax.ShapeDtypeStruct((B, D), jnp.float32),
        scratch_types=[
            pltpu.VMEM((b_per_w,), jnp.int32),
            pltpu.VMEM((b_per_w, D), jnp.float32),
            pltpu.SemaphoreType.DMA,
        ],
    )
    def gather_kernel(xs_hbm, w_hbm, out_hbm, idx_v, rows_v, sem):
        wid = lax.axis_index("s") * nc + lax.axis_index("c")
        base = wid * b_per_w
        pltpu.sync_copy(xs_hbm.at[pl.ds(base, b_per_w)], idx_v)
        pltpu.async_copy(w_hbm.at[idx_v], rows_v, sem).wait()
        pltpu.sync_copy(rows_v, out_hbm.at[pl.ds(base, b_per_w)])

    return gather_kernel(xs, weight)


def _tc_project(emb, weight, cols, batch_block=128, vocab_block=8192, ring=4):
    B, D = emb.shape
    V = weight.shape[0]
    BB, VB, S = batch_block, vocab_block, ring
    nvb = pl.cdiv(cols, VB)
    nbb = B // BB
    tail = cols - (nvb - 1) * VB
    assert tail % 128 == 0 and nbb >= S
    nsteps = nvb * nbb
    full_upto = (nvb - 1) * nbb

    def body(emb_ref, w_ref, out_hbm, bufs, sems):
        i = pl.program_id(0)
        j = pl.program_id(1)
        g = i * nbb + j
        b = lax.rem(g, S)

        def out_slice(gg, width):
            ii = gg // nbb
            jj = lax.rem(gg, nbb)
            return out_hbm.at[pl.ds(jj * BB, BB), pl.ds(ii * VB, width)]

        prev = g - S

        @pl.when(jnp.logical_and(prev >= 0, prev < full_upto))
        def _():
            pltpu.make_async_copy(bufs.at[b], out_slice(prev, VB),
                                  sems.at[b]).wait()

        @pl.when(prev >= full_upto)
        def _():
            pltpu.make_async_copy(bufs.at[b, :, pl.ds(0, tail)],
                                  out_slice(prev, tail), sems.at[b]).wait()

        a = emb_ref[...].astype(jnp.bfloat16)
        w = w_ref[...].astype(jnp.bfloat16)
        bufs[b] = lax.dot_general(
            a, w, (((1,), (1,)), ((), ())),
            preferred_element_type=jnp.float32,
        )

        nbands = BB // 8

        def start_tile(k, width):
            jj = lax.rem(g, nbb)
            ii = g // nbb
            for t in range(nbands):
                pltpu.make_async_copy(
                    bufs.at[k, pl.ds(t * 8, 8), pl.ds(0, width)],
                    out_hbm.at[pl.ds(jj * BB + t * 8, 8),
                               pl.ds(ii * VB, width)],
                    sems.at[k],
                ).start(priority=t % 2)

        for k in range(S):
            @pl.when(jnp.logical_and(b == k, g < full_upto))
            def _(k=k):
                start_tile(k, VB)

            @pl.when(jnp.logical_and(b == k, g >= full_upto))
            def _(k=k):
                start_tile(k, tail)

        @pl.when(g == nsteps - 1)
        def _():
            for k in range(S):
                gk = nsteps - 1 - k
                bk = gk % S
                pltpu.make_async_copy(bufs.at[bk, :, pl.ds(0, tail)],
                                      out_slice(gk, tail), sems.at[bk]).wait()

    return pl.pallas_call(
        body,
        grid=(nvb, nbb),
        in_specs=[
            pl.BlockSpec((BB, D), lambda i, j: (j, 0)),
            pl.BlockSpec((VB, D), lambda i, j: (i, 0)),
        ],
        out_specs=pl.BlockSpec(memory_space=pl.ANY),
        out_shape=jax.ShapeDtypeStruct((B, V), jnp.float32),
        scratch_shapes=[
            pltpu.VMEM((S, BB, VB), jnp.float32),
            pltpu.SemaphoreType.DMA((S,)),
        ],
    )(emb, weight)


def _tc_tail(emb, weight, col0, width):
    B, D = emb.shape

    def body(emb_ref, w_ref, out_ref):
        a = emb_ref[...].astype(jnp.bfloat16)
        w = w_ref[...].astype(jnp.bfloat16)
        out_ref[...] = lax.dot_general(
            a, w, (((1,), (1,)), ((), ())),
            preferred_element_type=jnp.float32,
        )

    return pl.pallas_call(
        body,
        grid=(1,),
        in_specs=[
            pl.BlockSpec((B, D), lambda i: (0, 0)),
            pl.BlockSpec((width, D), lambda i: (col0 // width, 0)),
        ],
        out_specs=pl.BlockSpec((B, width), lambda i: (0, 0)),
        out_shape=jax.ShapeDtypeStruct((B, width), jnp.float32),
    )(emb, weight)


def kernel(xs, weight):
    B = xs.shape[0]
    V = weight.shape[0]
    cols = (V // 128) * 128
    emb = _sc_gather(xs.astype(jnp.int32), weight)
    main = _tc_project(emb, weight, cols)
    tail = _tc_tail(emb, weight, cols, V - cols)
    return lax.dynamic_update_slice(main, tail, (0, cols))


# speedup vs baseline: 2.8699x; 2.8699x over previous
import functools

import jax
import jax.numpy as jnp
from jax import lax
from jax.experimental import pallas as pl
from jax.experimental.pallas import tpu as pltpu
from jax.experimental.pallas import tpu_sc as plsc


def _sc_gather(xs, weight):
    B = xs.shape[0]
    V, D = weight.shape
    info = plsc.get_sparse_core_info()
    nc, ns = info.num_cores, info.num_subcores
    nw = nc * ns
    b_per_w = B // nw

    mesh = plsc.VectorSubcoreMesh(core_axis_name="c", subcore_axis_name="s")

    @functools.partial(
        pl.kernel,
        mesh=mesh,
        out_type=jax.ShapeDtypeStruct((B, D), jnp.float32),
        scratch_types=[
            pltpu.VMEM((b_per_w,), jnp.int32),
            pltpu.VMEM((b_per_w, D), jnp.float32),
            pltpu.SemaphoreType.DMA,
        ],
    )
    def gather_kernel(xs_hbm, w_hbm, out_hbm, idx_v, rows_v, sem):
        wid = lax.axis_index("s") * nc + lax.axis_index("c")
        base = wid * b_per_w
        pltpu.sync_copy(xs_hbm.at[pl.ds(base, b_per_w)], idx_v)
        pltpu.async_copy(w_hbm.at[idx_v], rows_v, sem).wait()
        pltpu.sync_copy(rows_v, out_hbm.at[pl.ds(base, b_per_w)])

    return gather_kernel(xs, weight)


def _tc_project_t(emb, weight, vocab_block=2000, ring=4):
    B, D = emb.shape
    V = weight.shape[0]
    VB, S = vocab_block, ring
    assert V % VB == 0 and VB % 8 == 0
    nsteps = V // VB

    def body(emb_ref, w_ref, out_hbm, bufs, sems):
        g = pl.program_id(0)
        b = lax.rem(g, S)

        @pl.when(g >= S)
        def _():
            pltpu.make_async_copy(bufs.at[b], out_hbm.at[pl.ds((g - S) * VB, VB)],
                                  sems.at[b]).wait()

        a = emb_ref[...].astype(jnp.bfloat16)
        w = w_ref[...].astype(jnp.bfloat16)
        bufs[b] = lax.dot_general(
            w, a, (((1,), (1,)), ((), ())),
            preferred_element_type=jnp.float32,
        )

        for k in range(S):
            @pl.when(b == k)
            def _(k=k):
                pltpu.make_async_copy(bufs.at[k],
                                      out_hbm.at[pl.ds(g * VB, VB)],
                                      sems.at[k]).start(priority=k % 2)

        @pl.when(g == nsteps - 1)
        def _():
            for k in range(S):
                gk = nsteps - 1 - k
                bk = gk % S
                pltpu.make_async_copy(bufs.at[bk],
                                      out_hbm.at[pl.ds(gk * VB, VB)],
                                      sems.at[bk]).wait()

    return pl.pallas_call(
        body,
        grid=(nsteps,),
        in_specs=[
            pl.BlockSpec((B, D), lambda i: (0, 0)),
            pl.BlockSpec((VB, D), lambda i: (i, 0)),
        ],
        out_specs=pl.BlockSpec(memory_space=pl.ANY),
        out_shape=jax.ShapeDtypeStruct((V, B), jnp.float32),
        scratch_shapes=[
            pltpu.VMEM((S, VB, B), jnp.float32),
            pltpu.SemaphoreType.DMA((S,)),
        ],
    )(emb, weight)


def kernel(xs, weight):
    emb = _sc_gather(xs.astype(jnp.int32), weight)
    out_t = _tc_project_t(emb, weight)
    return out_t.T
